# baseline (speedup 1.0000x reference)
import functools

import jax
import jax.numpy as jnp
from jax import lax
from jax.experimental import pallas as pl
from jax.experimental.pallas import tpu as pltpu
from jax.experimental.pallas import tpu_sc as plsc

N_NODES = 10000
N_PAD = 10240
D = 128
E = 320000
NC = 2
NS = 16
NW = NC * NS
CH = 80
CPT = 125
NBUF = 4
NOUT = (CPT - 1) // NBUF
ROWS_PER_TILE = N_PAD // NS
LANES = 16
GROUPS = D // LANES


@functools.partial(
    pl.kernel,
    mesh=plsc.VectorSubcoreMesh(core_axis_name="c", subcore_axis_name="s"),
    out_type=[jax.ShapeDtypeStruct((N_PAD, D), jnp.float32),
              jax.ShapeDtypeStruct((N_PAD, D), jnp.float32)],
    scratch_types=[
        pltpu.VMEM_SHARED((N_PAD, D), jnp.float32),
        pltpu.VMEM((CH, D), jnp.float32),
        pltpu.VMEM((CH, D), jnp.float32),
        pltpu.VMEM((CH, D), jnp.float32),
        pltpu.VMEM((CH, D), jnp.float32),
        pltpu.VMEM((3 * CH,), jnp.int32),
        pltpu.VMEM((3 * CH,), jnp.int32),
        pltpu.VMEM((3 * CH,), jnp.int32),
        pltpu.VMEM((3 * CH,), jnp.int32),
        pltpu.VMEM((CH,), jnp.int32),
        pltpu.VMEM((CH,), jnp.int32),
        pltpu.VMEM((CH,), jnp.int32),
        pltpu.VMEM((CH,), jnp.int32),
        pltpu.SemaphoreType.DMA,
        pltpu.SemaphoreType.DMA,
        pltpu.SemaphoreType.DMA,
        pltpu.SemaphoreType.DMA,
        pltpu.SemaphoreType.DMA,
        pltpu.SemaphoreType.DMA,
        pltpu.SemaphoreType.DMA,
        pltpu.SemaphoreType.DMA,
        pltpu.SemaphoreType.DMA,
        pltpu.SemaphoreType.DMA,
        pltpu.SemaphoreType.DMA,
        pltpu.SemaphoreType.DMA,
    ],
)
def _sc_aggregate(packed_hbm, emb_hbm, out0_hbm, out1_hbm,
                  acc, rows0, rows1, rows2, rows3,
                  eb0, eb1, eb2, eb3, rib0, rib1, rib2, rib3,
                  g0, g1, g2, g3, s0, s1, s2, s3, i0, i1, i2, i3):
    c = lax.axis_index("c")
    s = lax.axis_index("s")
    wid = s * NC + c
    rows = (rows0, rows1, rows2, rows3)
    ebuf = (eb0, eb1, eb2, eb3)
    rib = (rib0, rib1, rib2, rib3)
    gsem = (g0, g1, g2, g3)
    ssem = (s0, s1, s2, s3)
    isem = (i0, i1, i2, i3)

    def zrow(i, carry):
        for g in range(GROUPS):
            rows0[i, pl.ds(g * LANES, LANES)] = jnp.zeros((LANES,), jnp.float32)
        return carry
    lax.fori_loop(0, CH, zrow, None)
    for j in range(ROWS_PER_TILE // CH):
        pltpu.sync_copy(rows0, acc.at[pl.ds(s * ROWS_PER_TILE + j * CH, CH)])
    plsc.subcore_barrier()

    cbase = wid * CPT * 3 * CH

    def start_idx(k, b):
        off = cbase + k * 3 * CH
        pltpu.async_copy(packed_hbm.at[pl.ds(off, 3 * CH)], ebuf[b], isem[b])

    def wait_idx(b):
        pltpu.make_async_copy(packed_hbm.at[pl.ds(0, 3 * CH)], ebuf[b],
                              isem[b]).wait()

    def start_gather(b):
        pltpu.async_copy(emb_hbm.at[ebuf[b].at[pl.ds(CH, CH)]], rows[b], gsem[b])

    def wait_gather(b):
        pltpu.make_async_copy(emb_hbm.at[ebuf[b].at[pl.ds(CH, CH)]], rows[b],
                              gsem[b]).wait()

    def start_scatter(b):
        pltpu.async_copy(rows[b], acc.at[rib[b]], ssem[b], add=True)

    def wait_scatter(b):
        pltpu.make_async_copy(rows[b], acc.at[rib[b]], ssem[b]).wait()

    def scale_chunk(b):
        rw = rows[b]
        eb = ebuf[b]
        for g5 in range(CH // LANES):
            rib[b][pl.ds(g5 * LANES, LANES)] = eb[pl.ds(g5 * LANES, LANES)]

        def scale16(j2, carry):
            wv16 = lax.bitcast_convert_type(
                eb[pl.ds(2 * CH + j2 * LANES, LANES)], jnp.float32)
            for e in range(LANES):
                wvec = jnp.full((LANES,), wv16[e], jnp.float32)
                i = j2 * LANES + e
                for g in range(GROUPS):
                    sl = pl.ds(g * LANES, LANES)
                    rw[i, sl] = rw[i, sl] * wvec
            return carry
        lax.fori_loop(0, CH // LANES, scale16, None)

    for b in range(NBUF - 1):
        start_idx(b, b)
    for b in range(2):
        wait_idx(b)
        start_gather(b)

    def chunk_iter(j, carry):
        for b in range(NBUF):
            k = j * NBUF + b
            p = (b + 3) % NBUF
            q = (b + 2) % NBUF
            wait_gather(b)
            scale_chunk(b)
            start_scatter(b)

            @pl.when(k >= 1)
            def _():
                wait_scatter(p)

            @pl.when(k < CPT - 3)
            def _():
                start_idx(k + 3, p)

            @pl.when(k < CPT - 2)
            def _():
                wait_idx(q)
                start_gather(q)
        return carry
    lax.fori_loop(0, NOUT, chunk_iter, None)

    wait_gather(0)
    scale_chunk(0)
    start_scatter(0)
    wait_scatter(3)
    wait_scatter(0)

    plsc.subcore_barrier()

    @pl.when(c == 0)
    def _():
        pltpu.sync_copy(acc.at[pl.ds(s * ROWS_PER_TILE, ROWS_PER_TILE)],
                        out0_hbm.at[pl.ds(s * ROWS_PER_TILE, ROWS_PER_TILE)])

    @pl.when(c == 1)
    def _():
        pltpu.sync_copy(acc.at[pl.ds(s * ROWS_PER_TILE, ROWS_PER_TILE)],
                        out1_hbm.at[pl.ds(s * ROWS_PER_TILE, ROWS_PER_TILE)])


def _mlp_body(p0, p1, w1, w2, o):
    x = p0[...] + p1[...]
    h = lax.dot_general(x, w1[...], (((1,), (1,)), ((), ())),
                        preferred_element_type=jnp.float32)
    h = jnp.maximum(h, 0.0)
    o[...] = lax.dot_general(h, w2[...], (((1,), (1,)), ((), ())),
                             preferred_element_type=jnp.float32)


def _tc_mlp(p0, p1, W1, W2):
    blk = 1000
    return pl.pallas_call(
        _mlp_body,
        grid=(N_NODES // blk,),
        in_specs=[
            pl.BlockSpec((blk, D), lambda i: (i, 0)),
            pl.BlockSpec((blk, D), lambda i: (i, 0)),
            pl.BlockSpec((D, D), lambda i: (0, 0)),
            pl.BlockSpec((D, D), lambda i: (0, 0)),
        ],
        out_specs=pl.BlockSpec((blk, D), lambda i: (i, 0)),
        out_shape=jax.ShapeDtypeStruct((N_NODES, D), jnp.float32),
    )(p0, p1, W1, W2)


def kernel(edge_index, edge_weight, emb_weight, W1, W2):
    nck = E // CH
    packed = jnp.concatenate([
        edge_index[0].reshape(nck, 1, CH),
        edge_index[1].reshape(nck, 1, CH),
        lax.bitcast_convert_type(edge_weight, jnp.int32).reshape(nck, 1, CH),
    ], axis=1).reshape(3 * E)
    p0, p1 = _sc_aggregate(packed, emb_weight)
    return _tc_mlp(p0, p1, W1, W2)


# speedup vs baseline: 1.3245x; 1.3245x over previous
import functools

import jax
import jax.numpy as jnp
from jax import lax
from jax.experimental import pallas as pl
from jax.experimental.pallas import tpu as pltpu
from jax.experimental.pallas import tpu_sc as plsc

N_NODES = 10000
N_PAD = 10240
D = 128
E = 320000
NC = 2
NS = 16
NW = NC * NS
CH = 80
CPT = 125
NBUF = 4
NOUT = (CPT - 1) // NBUF
ROWS_PER_TILE = N_PAD // NS
LANES = 16
GROUPS = D // LANES


@functools.partial(
    pl.kernel,
    mesh=plsc.VectorSubcoreMesh(core_axis_name="c", subcore_axis_name="s"),
    out_type=[jax.ShapeDtypeStruct((N_PAD, D), jnp.float32),
              jax.ShapeDtypeStruct((N_PAD, D), jnp.float32)],
    scratch_types=[
        pltpu.VMEM_SHARED((N_PAD, D), jnp.float32),
        pltpu.VMEM((CH, D), jnp.float32),
        pltpu.VMEM((CH, D), jnp.float32),
        pltpu.VMEM((CH, D), jnp.float32),
        pltpu.VMEM((CH, D), jnp.float32),
        pltpu.VMEM((CH,), jnp.int32),
        pltpu.VMEM((CH,), jnp.int32),
        pltpu.VMEM((CH,), jnp.int32),
        pltpu.VMEM((CH,), jnp.int32),
        pltpu.VMEM((CH,), jnp.int32),
        pltpu.VMEM((CH,), jnp.int32),
        pltpu.VMEM((CH,), jnp.int32),
        pltpu.VMEM((CH,), jnp.int32),
        pltpu.VMEM((CH,), jnp.float32),
        pltpu.VMEM((CH,), jnp.float32),
        pltpu.VMEM((CH,), jnp.float32),
        pltpu.VMEM((CH,), jnp.float32),
        pltpu.SemaphoreType.DMA,
        pltpu.SemaphoreType.DMA,
        pltpu.SemaphoreType.DMA,
        pltpu.SemaphoreType.DMA,
        pltpu.SemaphoreType.DMA,
        pltpu.SemaphoreType.DMA,
        pltpu.SemaphoreType.DMA,
        pltpu.SemaphoreType.DMA,
        pltpu.SemaphoreType.DMA,
        pltpu.SemaphoreType.DMA,
        pltpu.SemaphoreType.DMA,
        pltpu.SemaphoreType.DMA,
    ],
)
def _sc_aggregate(edges_hbm, w_hbm, emb_hbm, out0_hbm, out1_hbm,
                  acc, rows0, rows1, rows2, rows3,
                  rib0, rib1, rib2, rib3, cib0, cib1, cib2, cib3,
                  wvb0, wvb1, wvb2, wvb3,
                  g0, g1, g2, g3, s0, s1, s2, s3, i0, i1, i2, i3):
    c = lax.axis_index("c")
    s = lax.axis_index("s")
    wid = s * NC + c
    rows = (rows0, rows1, rows2, rows3)
    rib = (rib0, rib1, rib2, rib3)
    cib = (cib0, cib1, cib2, cib3)
    wvb = (wvb0, wvb1, wvb2, wvb3)
    gsem = (g0, g1, g2, g3)
    ssem = (s0, s1, s2, s3)
    isem = (i0, i1, i2, i3)

    def zrow(i, carry):
        for g in range(GROUPS):
            rows0[i, pl.ds(g * LANES, LANES)] = jnp.zeros((LANES,), jnp.float32)
        return carry
    lax.fori_loop(0, CH, zrow, None)
    for j in range(ROWS_PER_TILE // CH):
        pltpu.sync_copy(rows0, acc.at[pl.ds(s * ROWS_PER_TILE + j * CH, CH)])
    plsc.subcore_barrier()

    cbase = wid * CPT * CH

    def start_idx(k, b):
        off = cbase + k * CH
        pltpu.async_copy(edges_hbm.at[pl.ds(E + off, CH)], cib[b], isem[b])
        pltpu.async_copy(edges_hbm.at[pl.ds(off, CH)], rib[b], isem[b])
        pltpu.async_copy(w_hbm.at[pl.ds(off, CH)], wvb[b], isem[b])

    def wait_idx(b):
        pltpu.make_async_copy(edges_hbm.at[pl.ds(0, CH)], cib[b], isem[b]).wait()
        pltpu.make_async_copy(edges_hbm.at[pl.ds(0, CH)], rib[b], isem[b]).wait()
        pltpu.make_async_copy(w_hbm.at[pl.ds(0, CH)], wvb[b], isem[b]).wait()

    def start_gather(b):
        pltpu.async_copy(emb_hbm.at[cib[b]], rows[b], gsem[b])

    def wait_gather(b):
        pltpu.make_async_copy(emb_hbm.at[cib[b]], rows[b], gsem[b]).wait()

    def start_scatter(b):
        pltpu.async_copy(rows[b], acc.at[rib[b]], ssem[b], add=True)

    def wait_scatter(b):
        pltpu.make_async_copy(rows[b], acc.at[rib[b]], ssem[b]).wait()

    def scale_chunk(b):
        rw = rows[b]
        wref = wvb[b]

        def scale16(j2, carry):
            wv16 = wref[pl.ds(j2 * LANES, LANES)]
            for e in range(LANES):
                wvec = jnp.full((LANES,), wv16[e], jnp.float32)
                i = j2 * LANES + e
                for g in range(GROUPS):
                    sl = pl.ds(g * LANES, LANES)
                    rw[i, sl] = rw[i, sl] * wvec
            return carry
        lax.fori_loop(0, CH // LANES, scale16, None)

    for b in range(NBUF - 1):
        start_idx(b, b)
    for b in range(2):
        wait_idx(b)
        start_gather(b)

    def chunk_iter(j, carry):
        for b in range(NBUF):
            k = j * NBUF + b
            p = (b + 3) % NBUF
            q = (b + 2) % NBUF
            wait_gather(b)
            scale_chunk(b)
            start_scatter(b)

            @pl.when(k >= 1)
            def _():
                wait_scatter(p)

            @pl.when(k < CPT - 3)
            def _():
                start_idx(k + 3, p)

            @pl.when(k < CPT - 2)
            def _():
                wait_idx(q)
                start_gather(q)
        return carry
    lax.fori_loop(0, NOUT, chunk_iter, None)

    wait_gather(0)
    scale_chunk(0)
    start_scatter(0)
    wait_scatter(3)
    wait_scatter(0)

    plsc.subcore_barrier()

    @pl.when(c == 0)
    def _():
        pltpu.sync_copy(acc.at[pl.ds(s * ROWS_PER_TILE, ROWS_PER_TILE)],
                        out0_hbm.at[pl.ds(s * ROWS_PER_TILE, ROWS_PER_TILE)])

    @pl.when(c == 1)
    def _():
        pltpu.sync_copy(acc.at[pl.ds(s * ROWS_PER_TILE, ROWS_PER_TILE)],
                        out1_hbm.at[pl.ds(s * ROWS_PER_TILE, ROWS_PER_TILE)])


def _mlp_body(p0, p1, w1, w2, o):
    x = p0[...] + p1[...]
    h = lax.dot_general(x, w1[...], (((1,), (1,)), ((), ())),
                        preferred_element_type=jnp.float32)
    h = jnp.maximum(h, 0.0)
    o[...] = lax.dot_general(h, w2[...], (((1,), (1,)), ((), ())),
                             preferred_element_type=jnp.float32)


def _tc_mlp(p0, p1, W1, W2):
    blk = 1000
    return pl.pallas_call(
        _mlp_body,
        grid=(N_NODES // blk,),
        in_specs=[
            pl.BlockSpec((blk, D), lambda i: (i, 0)),
            pl.BlockSpec((blk, D), lambda i: (i, 0)),
            pl.BlockSpec((D, D), lambda i: (0, 0)),
            pl.BlockSpec((D, D), lambda i: (0, 0)),
        ],
        out_specs=pl.BlockSpec((blk, D), lambda i: (i, 0)),
        out_shape=jax.ShapeDtypeStruct((N_NODES, D), jnp.float32),
    )(p0, p1, W1, W2)


def kernel(edge_index, edge_weight, emb_weight, W1, W2):
    p0, p1 = _sc_aggregate(edge_index.reshape(2 * E), edge_weight, emb_weight)
    return _tc_mlp(p0, p1, W1, W2)


# speedup vs baseline: 1.5489x; 1.1694x over previous
import functools

import jax
import jax.numpy as jnp
from jax import lax
from jax.experimental import pallas as pl
from jax.experimental.pallas import tpu as pltpu
from jax.experimental.pallas import tpu_sc as plsc

N_NODES = 10000
N_PAD = 10240
D = 128
E = 320000
NC = 2
NS = 16
NW = NC * NS
CH = 80
CPT = 125
NBUF = 4
NOUT = (CPT - 1) // NBUF
ROWS_PER_TILE = N_PAD // NS
LANES = 16
GROUPS = D // LANES


@functools.partial(
    pl.kernel,
    mesh=plsc.VectorSubcoreMesh(core_axis_name="c", subcore_axis_name="s"),
    out_type=[jax.ShapeDtypeStruct((N_PAD, D), jnp.float32),
              jax.ShapeDtypeStruct((N_PAD, D), jnp.float32)],
    scratch_types=[
        pltpu.VMEM_SHARED((N_PAD, D), jnp.float32),
        pltpu.VMEM((CH, D), jnp.float32),
        pltpu.VMEM((CH, D), jnp.float32),
        pltpu.VMEM((CH, D), jnp.float32),
        pltpu.VMEM((CH, D), jnp.float32),
        pltpu.VMEM((CH,), jnp.int32),
        pltpu.VMEM((CH,), jnp.int32),
        pltpu.VMEM((CH,), jnp.int32),
        pltpu.VMEM((CH,), jnp.int32),
        pltpu.VMEM((CH,), jnp.int32),
        pltpu.VMEM((CH,), jnp.int32),
        pltpu.VMEM((CH,), jnp.int32),
        pltpu.VMEM((CH,), jnp.int32),
        pltpu.VMEM((CH,), jnp.float32),
        pltpu.VMEM((CH,), jnp.float32),
        pltpu.VMEM((CH,), jnp.float32),
        pltpu.VMEM((CH,), jnp.float32),
        pltpu.SemaphoreType.DMA,
        pltpu.SemaphoreType.DMA,
        pltpu.SemaphoreType.DMA,
        pltpu.SemaphoreType.DMA,
        pltpu.SemaphoreType.DMA,
        pltpu.SemaphoreType.DMA,
        pltpu.SemaphoreType.DMA,
        pltpu.SemaphoreType.DMA,
        pltpu.SemaphoreType.DMA,
        pltpu.SemaphoreType.DMA,
        pltpu.SemaphoreType.DMA,
        pltpu.SemaphoreType.DMA,
    ],
)
def _sc_aggregate(edges_hbm, w_hbm, emb_hbm, out0_hbm, out1_hbm,
                  acc, rows0, rows1, rows2, rows3,
                  rib0, rib1, rib2, rib3, cib0, cib1, cib2, cib3,
                  wvb0, wvb1, wvb2, wvb3,
                  g0, g1, g2, g3, s0, s1, s2, s3, i0, i1, i2, i3):
    c = lax.axis_index("c")
    s = lax.axis_index("s")
    wid = s * NC + c
    rows = (rows0, rows1, rows2, rows3)
    rib = (rib0, rib1, rib2, rib3)
    cib = (cib0, cib1, cib2, cib3)
    wvb = (wvb0, wvb1, wvb2, wvb3)
    gsem = (g0, g1, g2, g3)
    ssem = (s0, s1, s2, s3)
    isem = (i0, i1, i2, i3)

    def zrow(i, carry):
        for g in range(GROUPS):
            rows0[i, pl.ds(g * LANES, LANES)] = jnp.zeros((LANES,), jnp.float32)
        return carry
    lax.fori_loop(0, CH, zrow, None)
    for j in range(ROWS_PER_TILE // CH):
        pltpu.sync_copy(rows0, acc.at[pl.ds(s * ROWS_PER_TILE + j * CH, CH)])
    plsc.subcore_barrier()

    cbase = wid * CPT * CH

    def start_idx(k, b):
        off = cbase + k * CH
        pltpu.async_copy(edges_hbm.at[pl.ds(E + off, CH)], cib[b], isem[b])
        pltpu.async_copy(edges_hbm.at[pl.ds(off, CH)], rib[b], isem[b])
        pltpu.async_copy(w_hbm.at[pl.ds(off, CH)], wvb[b], isem[b])

    def wait_idx(b):
        pltpu.make_async_copy(edges_hbm.at[pl.ds(0, CH)], cib[b], isem[b]).wait()
        pltpu.make_async_copy(edges_hbm.at[pl.ds(0, CH)], rib[b], isem[b]).wait()
        pltpu.make_async_copy(w_hbm.at[pl.ds(0, CH)], wvb[b], isem[b]).wait()

    def start_gather(b):
        pltpu.async_copy(emb_hbm.at[cib[b]], rows[b], gsem[b])

    def wait_gather(b):
        pltpu.make_async_copy(emb_hbm.at[cib[b]], rows[b], gsem[b]).wait()

    def start_scatter(b):
        pltpu.async_copy(rows[b], acc.at[rib[b]], ssem[b], add=True)

    def wait_scatter(b):
        pltpu.make_async_copy(rows[b], acc.at[rib[b]], ssem[b]).wait()

    def scale_chunk(b):
        rw = rows[b]
        wref = wvb[b]

        @functools.partial(plsc.parallel_loop, 0, CH // LANES, unroll=2)
        def scale16(j2):
            wv16 = wref[pl.ds(j2 * LANES, LANES)]
            for e in range(LANES):
                wvec = jnp.full((LANES,), wv16[e], jnp.float32)
                i = j2 * LANES + e
                for g in range(GROUPS):
                    sl = pl.ds(g * LANES, LANES)
                    rw[i, sl] = rw[i, sl] * wvec

    for b in range(NBUF - 1):
        start_idx(b, b)
    for b in range(2):
        wait_idx(b)
        start_gather(b)

    def chunk_iter(j, carry):
        for b in range(NBUF):
            k = j * NBUF + b
            p = (b + 3) % NBUF
            q = (b + 2) % NBUF
            wait_gather(b)
            scale_chunk(b)
            start_scatter(b)

            @pl.when(k >= 1)
            def _():
                wait_scatter(p)

            @pl.when(k < CPT - 3)
            def _():
                start_idx(k + 3, p)

            @pl.when(k < CPT - 2)
            def _():
                wait_idx(q)
                start_gather(q)
        return carry
    lax.fori_loop(0, NOUT, chunk_iter, None)

    wait_gather(0)
    scale_chunk(0)
    start_scatter(0)
    wait_scatter(3)
    wait_scatter(0)

    plsc.subcore_barrier()

    @pl.when(c == 0)
    def _():
        pltpu.sync_copy(acc.at[pl.ds(s * ROWS_PER_TILE, ROWS_PER_TILE)],
                        out0_hbm.at[pl.ds(s * ROWS_PER_TILE, ROWS_PER_TILE)])

    @pl.when(c == 1)
    def _():
        pltpu.sync_copy(acc.at[pl.ds(s * ROWS_PER_TILE, ROWS_PER_TILE)],
                        out1_hbm.at[pl.ds(s * ROWS_PER_TILE, ROWS_PER_TILE)])


def _mlp_body(p0, p1, w1, w2, o):
    x = p0[...] + p1[...]
    h = lax.dot_general(x, w1[...], (((1,), (1,)), ((), ())),
                        preferred_element_type=jnp.float32)
    h = jnp.maximum(h, 0.0)
    o[...] = lax.dot_general(h, w2[...], (((1,), (1,)), ((), ())),
                             preferred_element_type=jnp.float32)


def _tc_mlp(p0, p1, W1, W2):
    blk = 1000
    return pl.pallas_call(
        _mlp_body,
        grid=(N_NODES // blk,),
        in_specs=[
            pl.BlockSpec((blk, D), lambda i: (i, 0)),
            pl.BlockSpec((blk, D), lambda i: (i, 0)),
            pl.BlockSpec((D, D), lambda i: (0, 0)),
            pl.BlockSpec((D, D), lambda i: (0, 0)),
        ],
        out_specs=pl.BlockSpec((blk, D), lambda i: (i, 0)),
        out_shape=jax.ShapeDtypeStruct((N_NODES, D), jnp.float32),
    )(p0, p1, W1, W2)


def kernel(edge_index, edge_weight, emb_weight, W1, W2):
    p0, p1 = _sc_aggregate(edge_index.reshape(2 * E), edge_weight, emb_weight)
    return _tc_mlp(p0, p1, W1, W2)
